# baseline (speedup 1.0000x reference)
import jax
import jax.numpy as jnp
from jax import lax
from jax.experimental import pallas as pl
from jax.experimental.pallas import tpu as pltpu
from jax.experimental.pallas import tpu_sc as plsc

_N = 10000
_F = 128
_NC = 2
_NS = 16
_NW = _NC * _NS
_CH = 128
_NCHUNK = 80
_EPW = _CH * _NCHUNK
_NC0 = 96
_NC1 = 64
_NSTAGE = max(_NC0, _NC1)
_NPAD = 10240
_RPT = _NPAD // _NS

_MESH = plsc.VectorSubcoreMesh(core_axis_name="c", subcore_axis_name="s")


def _deg_body(dst_hbm, zeros_hbm, ones_hbm, out_hbm, dvm, ones_vm, acc):
    c = lax.axis_index("c")
    s = lax.axis_index("s")
    w = c * _NS + s
    pltpu.sync_copy(zeros_hbm, acc.at[pl.ds(s * _RPT, _RPT)])
    pltpu.sync_copy(ones_hbm, ones_vm)
    pltpu.sync_copy(dst_hbm.at[w], dvm)
    plsc.subcore_barrier()

    @pl.loop(0, _NCHUNK)
    def _chunk(j):
        pltpu.sync_copy(ones_vm, acc.at[dvm.at[j]], add=True)

    plsc.subcore_barrier()
    pltpu.sync_copy(acc.at[pl.ds(s * _RPT, _RPT)],
                    out_hbm.at[c, pl.ds(s * _RPT, _RPT)])


_deg_call = pl.kernel(
    _deg_body,
    out_type=jax.ShapeDtypeStruct((_NC, _NPAD), jnp.float32),
    mesh=_MESH,
    scratch_types=[
        pltpu.VMEM((_NCHUNK, _CH), jnp.int32),
        pltpu.VMEM((_CH,), jnp.float32),
        pltpu.VMEM_SHARED((_NPAD,), jnp.float32),
    ],
)


def _msg_body(xs_hbm, src0_hbm, dst0_hbm, src1_hbm, dst1_hbm, zrows_hbm,
              out_hbm, svm, dvm, rb0, acc):
    c = lax.axis_index("c")
    s = lax.axis_index("s")
    pltpu.sync_copy(zrows_hbm, acc.at[pl.ds(s * _RPT, _RPT)])
    plsc.subcore_barrier()

    @pl.when(c == 0)
    def _core0():
        pltpu.sync_copy(src0_hbm.at[s], svm.at[pl.ds(0, _NC0)])
        pltpu.sync_copy(dst0_hbm.at[s], dvm.at[pl.ds(0, _NC0)])

        @pl.loop(0, _NC0)
        def _chunk(j):
            pltpu.sync_copy(xs_hbm.at[svm.at[j]], rb0)
            pltpu.sync_copy(rb0, acc.at[dvm.at[j]], add=True)

    @pl.when(c == 1)
    def _core1():
        pltpu.sync_copy(src1_hbm.at[s], svm.at[pl.ds(0, _NC1)])
        pltpu.sync_copy(dst1_hbm.at[s], dvm.at[pl.ds(0, _NC1)])

        @pl.loop(0, _NC1)
        def _chunk(j):
            pltpu.sync_copy(xs_hbm.at[svm.at[j]], rb0)
            pltpu.sync_copy(rb0, acc.at[dvm.at[j]], add=True)

    plsc.subcore_barrier()
    pltpu.sync_copy(acc.at[pl.ds(s * _RPT, _RPT)],
                    out_hbm.at[c, pl.ds(s * _RPT, _RPT)])


_msg_call = pl.kernel(
    _msg_body,
    out_type=jax.ShapeDtypeStruct((_NC, _NPAD, _F), jnp.float32),
    mesh=_MESH,
    scratch_types=[
        pltpu.VMEM((_NSTAGE, _CH), jnp.int32),
        pltpu.VMEM((_NSTAGE, _CH), jnp.int32),
        pltpu.VMEM((_CH, _F), jnp.float32),
        pltpu.VMEM_SHARED((_NPAD, _F), jnp.float32),
    ],
)


_BLK = 1000


def _tc_pre_body(x_ref, w_ref, d0_ref, d1_ref, o_ref):
    dinv = lax.rsqrt(d0_ref[...] + d1_ref[...] + 1.0)
    o_ref[...] = jnp.dot(x_ref[...], w_ref[...],
                         preferred_element_type=jnp.float32) * dinv


def _tc_mid_body(a0_ref, a1_ref, xs_ref, d0_ref, d1_ref, b_ref, w_ref, o_ref):
    dinv = lax.rsqrt(d0_ref[...] + d1_ref[...] + 1.0)
    z = dinv * (a0_ref[...] + a1_ref[...] + xs_ref[...]) + b_ref[...]
    h = jnp.maximum(z, 0.0)
    o_ref[...] = jnp.dot(h, w_ref[...],
                         preferred_element_type=jnp.float32) * dinv


def _tc_fin_body(a0_ref, a1_ref, xs_ref, d0_ref, d1_ref, b_ref, o_ref):
    dinv = lax.rsqrt(d0_ref[...] + d1_ref[...] + 1.0)
    o_ref[...] = dinv * (a0_ref[...] + a1_ref[...] + xs_ref[...]) + b_ref[...]


def _row_spec():
    return pl.BlockSpec((_BLK, _F), lambda i: (i, 0))


def _deg_spec():
    return pl.BlockSpec((_BLK, 1), lambda i: (i, 0))


def _full_spec(shape):
    return pl.BlockSpec(shape, lambda i: (0, 0))


_tc_pre = pl.pallas_call(
    _tc_pre_body,
    grid=(_N // _BLK,),
    in_specs=[_row_spec(), _full_spec((_F, _F)), _deg_spec(), _deg_spec()],
    out_specs=_row_spec(),
    out_shape=jax.ShapeDtypeStruct((_N, _F), jnp.float32),
)

_tc_mid = pl.pallas_call(
    _tc_mid_body,
    grid=(_N // _BLK,),
    in_specs=[_row_spec(), _row_spec(), _row_spec(), _deg_spec(), _deg_spec(),
              _full_spec((1, _F)), _full_spec((_F, _F))],
    out_specs=_row_spec(),
    out_shape=jax.ShapeDtypeStruct((_N, _F), jnp.float32),
)

_tc_fin = pl.pallas_call(
    _tc_fin_body,
    grid=(_N // _BLK,),
    in_specs=[_row_spec(), _row_spec(), _row_spec(), _deg_spec(), _deg_spec(),
              _full_spec((1, _F))],
    out_specs=_row_spec(),
    out_shape=jax.ShapeDtypeStruct((_N, _F), jnp.float32),
)


def kernel(x, edge_index, W1, b1, W2, b2):
    src = edge_index[0].astype(jnp.int32)
    dst = edge_index[1].astype(jnp.int32)
    e = src.shape[0]
    epad = _NW * _EPW
    src_p = jnp.concatenate(
        [src, jnp.zeros((epad - e,), jnp.int32)]).reshape(_NW, _NCHUNK, _CH)
    dst_p = jnp.concatenate(
        [dst, jnp.full((epad - e,), _N, jnp.int32)]).reshape(_NW, _NCHUNK, _CH)

    e0 = _NS * _NC0 * _CH
    emsg = e0 + _NS * _NC1 * _CH
    src_m = jnp.concatenate([src, jnp.zeros((emsg - e,), jnp.int32)])
    dst_m = jnp.concatenate([dst, jnp.full((emsg - e,), _N, jnp.int32)])
    src0 = src_m[:e0].reshape(_NS, _NC0, _CH)
    dst0 = dst_m[:e0].reshape(_NS, _NC0, _CH)
    src1 = src_m[e0:].reshape(_NS, _NC1, _CH)
    dst1 = dst_m[e0:].reshape(_NS, _NC1, _CH)

    zeros1 = jnp.zeros((_RPT,), jnp.float32)
    ones1 = jnp.ones((_CH,), jnp.float32)
    zrows = jnp.zeros((_RPT, _F), jnp.float32)
    b1r = b1.reshape(1, _F)
    b2r = b2.reshape(1, _F)

    deg2 = _deg_call(dst_p, zeros1, ones1)
    d0 = deg2[0, :_N].reshape(_N, 1)
    d1 = deg2[1, :_N].reshape(_N, 1)

    xs1 = _tc_pre(x, W1, d0, d1)
    acc1 = _msg_call(xs1, src0, dst0, src1, dst1, zrows)
    xs2 = _tc_mid(acc1[0, :_N], acc1[1, :_N], xs1, d0, d1, b1r, W2)
    acc2 = _msg_call(xs2, src0, dst0, src1, dst1, zrows)
    return _tc_fin(acc2[0, :_N], acc2[1, :_N], xs2, d0, d1, b2r)


# speedup vs baseline: 1.4945x; 1.4945x over previous
import jax
import jax.numpy as jnp
from jax import lax
from jax.experimental import pallas as pl
from jax.experimental.pallas import tpu as pltpu
from jax.experimental.pallas import tpu_sc as plsc

_N = 10000
_F = 128
_NC = 2
_NS = 16
_NW = _NC * _NS
_CH = 128
_NCHUNK = 79
_EPW = _CH * _NCHUNK
_NPAD = 10240
_RPT = _NPAD // _NS

_MESH = plsc.VectorSubcoreMesh(core_axis_name="c", subcore_axis_name="s")


def _deg_body(dst_hbm, zeros_hbm, ones_hbm, out_hbm, dvm, ones_vm, acc):
    c = lax.axis_index("c")
    s = lax.axis_index("s")
    w = c * _NS + s
    pltpu.sync_copy(zeros_hbm, acc.at[pl.ds(s * _RPT, _RPT)])
    pltpu.sync_copy(ones_hbm, ones_vm)
    pltpu.sync_copy(dst_hbm.at[w], dvm)
    plsc.subcore_barrier()

    @pl.loop(0, _NCHUNK)
    def _chunk(j):
        pltpu.sync_copy(ones_vm, acc.at[dvm.at[j]], add=True)

    plsc.subcore_barrier()
    pltpu.sync_copy(acc.at[pl.ds(s * _RPT, _RPT)],
                    out_hbm.at[c, pl.ds(s * _RPT, _RPT)])


_deg_call = pl.kernel(
    _deg_body,
    out_type=jax.ShapeDtypeStruct((_NC, _NPAD), jnp.float32),
    mesh=_MESH,
    scratch_types=[
        pltpu.VMEM((_NCHUNK, _CH), jnp.int32),
        pltpu.VMEM((_CH,), jnp.float32),
        pltpu.VMEM_SHARED((_NPAD,), jnp.float32),
    ],
)


def _msg_body(xs_hbm, src_hbm, dst_hbm, zrows_hbm, out_hbm,
              svm, dvm, rbuf, acc):
    c = lax.axis_index("c")
    s = lax.axis_index("s")
    w = c * _NS + s
    pltpu.sync_copy(zrows_hbm, acc.at[pl.ds(s * _RPT, _RPT)])
    pltpu.sync_copy(src_hbm.at[w], svm)
    pltpu.sync_copy(dst_hbm.at[w], dvm)
    plsc.subcore_barrier()

    @pl.loop(0, _NCHUNK)
    def _chunk(j):
        pltpu.sync_copy(xs_hbm.at[svm.at[j]], rbuf)
        pltpu.sync_copy(rbuf, acc.at[dvm.at[j]], add=True)

    plsc.subcore_barrier()
    pltpu.sync_copy(acc.at[pl.ds(s * _RPT, _RPT)],
                    out_hbm.at[c, pl.ds(s * _RPT, _RPT)])


_msg_call = pl.kernel(
    _msg_body,
    out_type=jax.ShapeDtypeStruct((_NC, _NPAD, _F), jnp.float32),
    mesh=_MESH,
    scratch_types=[
        pltpu.VMEM((_NCHUNK, _CH), jnp.int32),
        pltpu.VMEM((_NCHUNK, _CH), jnp.int32),
        pltpu.VMEM((_CH, _F), jnp.float32),
        pltpu.VMEM_SHARED((_NPAD, _F), jnp.float32),
    ],
)


_BLK = 1000


def _tc_pre_body(x_ref, w_ref, d0_ref, d1_ref, o_ref):
    dinv = lax.rsqrt(d0_ref[...] + d1_ref[...] + 1.0)
    o_ref[...] = jnp.dot(x_ref[...], w_ref[...],
                         preferred_element_type=jnp.float32) * dinv


def _tc_mid_body(a0_ref, a1_ref, xs_ref, d0_ref, d1_ref, b_ref, w_ref, o_ref):
    dinv = lax.rsqrt(d0_ref[...] + d1_ref[...] + 1.0)
    z = dinv * (a0_ref[...] + a1_ref[...] + xs_ref[...]) + b_ref[...]
    h = jnp.maximum(z, 0.0)
    o_ref[...] = jnp.dot(h, w_ref[...],
                         preferred_element_type=jnp.float32) * dinv


def _tc_fin_body(a0_ref, a1_ref, xs_ref, d0_ref, d1_ref, b_ref, o_ref):
    dinv = lax.rsqrt(d0_ref[...] + d1_ref[...] + 1.0)
    o_ref[...] = dinv * (a0_ref[...] + a1_ref[...] + xs_ref[...]) + b_ref[...]


def _row_spec():
    return pl.BlockSpec((_BLK, _F), lambda i: (i, 0))


def _deg_spec():
    return pl.BlockSpec((_BLK, 1), lambda i: (i, 0))


def _full_spec(shape):
    return pl.BlockSpec(shape, lambda i: (0, 0))


_tc_pre = pl.pallas_call(
    _tc_pre_body,
    grid=(_N // _BLK,),
    in_specs=[_row_spec(), _full_spec((_F, _F)), _deg_spec(), _deg_spec()],
    out_specs=_row_spec(),
    out_shape=jax.ShapeDtypeStruct((_N, _F), jnp.float32),
)

_tc_mid = pl.pallas_call(
    _tc_mid_body,
    grid=(_N // _BLK,),
    in_specs=[_row_spec(), _row_spec(), _row_spec(), _deg_spec(), _deg_spec(),
              _full_spec((1, _F)), _full_spec((_F, _F))],
    out_specs=_row_spec(),
    out_shape=jax.ShapeDtypeStruct((_N, _F), jnp.float32),
)

_tc_fin = pl.pallas_call(
    _tc_fin_body,
    grid=(_N // _BLK,),
    in_specs=[_row_spec(), _row_spec(), _row_spec(), _deg_spec(), _deg_spec(),
              _full_spec((1, _F))],
    out_specs=_row_spec(),
    out_shape=jax.ShapeDtypeStruct((_N, _F), jnp.float32),
)


def kernel(x, edge_index, W1, b1, W2, b2):
    src = edge_index[0].astype(jnp.int32)
    dst = edge_index[1].astype(jnp.int32)
    e = src.shape[0]
    epad = _NW * _EPW
    src_p = jnp.concatenate(
        [src, jnp.zeros((epad - e,), jnp.int32)]).reshape(_NW, _NCHUNK, _CH)
    dst_p = jnp.concatenate(
        [dst, jnp.full((epad - e,), _N, jnp.int32)]).reshape(_NW, _NCHUNK, _CH)

    zeros1 = jnp.zeros((_RPT,), jnp.float32)
    ones1 = jnp.ones((_CH,), jnp.float32)
    zrows = jnp.zeros((_RPT, _F), jnp.float32)
    b1r = b1.reshape(1, _F)
    b2r = b2.reshape(1, _F)

    deg2 = _deg_call(dst_p, zeros1, ones1)
    d0 = deg2[0, :_N].reshape(_N, 1)
    d1 = deg2[1, :_N].reshape(_N, 1)

    xs1 = _tc_pre(x, W1, d0, d1)
    acc1 = _msg_call(xs1, src_p, dst_p, zrows)
    xs2 = _tc_mid(acc1[0, :_N], acc1[1, :_N], xs1, d0, d1, b1r, W2)
    acc2 = _msg_call(xs2, src_p, dst_p, zrows)
    return _tc_fin(acc2[0, :_N], acc2[1, :_N], xs2, d0, d1, b2r)
